# baseline (speedup 1.0000x reference)
import functools

import jax
import jax.numpy as jnp
import numpy as np
from jax.experimental import pallas as pl
from jax.experimental.pallas import tpu as pltpu


def _compute_feats(object_grids, object_features, W1, b1, W2, b2, *, width):
    n_obj, g, _ = object_grids.shape
    n_pts = n_obj * g
    pos_dim = W1.shape[1] - object_features.shape[1]
    hid = W1.shape[0]
    out_dim = W2.shape[0]

    n_blocks = 32
    blk_obj = n_obj // n_blocks
    blk_pts = blk_obj * g

    ys = object_grids[:, :, 0].reshape(n_blocks, 1, blk_pts)
    xs = object_grids[:, :, 1].reshape(n_blocks, 1, blk_pts)
    oft = jnp.swapaxes(
        object_features.reshape(n_blocks, blk_obj, object_features.shape[1]),
        1, 2).reshape(n_blocks * object_features.shape[1], blk_obj)

    w1p = jnp.concatenate([W1[:, 0:pos_dim:2], W1[:, 1:pos_dim:2]], axis=1)
    w1o = W1[:, pos_dim:]
    div = np.exp(np.arange(0, pos_dim, 2, dtype=np.float64)
                 * -(np.log(10000.0) / pos_dim)).astype(np.float32)
    div = jnp.asarray(div).reshape(pos_dim // 2, 1)

    grid = (n_blocks,)
    kernel_fn = functools.partial(_feats_kernel_body, g=g, width=width)
    feats, idx = pl.pallas_call(
        kernel_fn,
        grid=grid,
        in_specs=[
            pl.BlockSpec((pos_dim // 2, 1), lambda i: (0, 0)),
            pl.BlockSpec((1, 1, blk_pts), lambda i: (i, 0, 0)),
            pl.BlockSpec((1, 1, blk_pts), lambda i: (i, 0, 0)),
            pl.BlockSpec((object_features.shape[1], blk_obj), lambda i: (i, 0)),
            pl.BlockSpec((hid, pos_dim), lambda i: (0, 0)),
            pl.BlockSpec((hid, object_features.shape[1]), lambda i: (0, 0)),
            pl.BlockSpec((hid, 1), lambda i: (0, 0)),
            pl.BlockSpec((out_dim, hid), lambda i: (0, 0)),
            pl.BlockSpec((1, out_dim), lambda i: (0, 0)),
        ],
        out_specs=[
            pl.BlockSpec((blk_pts, out_dim), lambda i: (i, 0)),
            pl.BlockSpec((1, 1, blk_pts), lambda i: (i, 0, 0)),
        ],
        out_shape=[
            jax.ShapeDtypeStruct((n_pts, out_dim), jnp.float32),
            jax.ShapeDtypeStruct((n_blocks, 1, blk_pts), jnp.int32),
        ],
    )(div, ys, xs, oft, w1p, w1o, b1.reshape(hid, 1), W2,
      b2.reshape(1, out_dim))
    return feats, idx.reshape(n_pts)


def _feats_kernel_body(div_ref, ys_ref, xs_ref, oft_ref, w1p_ref, w1o_ref,
                       b1_ref, w2_ref, b2_ref, feats_ref, idx_ref, *, g, width):
    y = ys_ref[0]
    x = xs_ref[0]
    yf = y.astype(jnp.float32)
    xf = x.astype(jnp.float32)
    div = div_ref[...]
    s_t = jnp.sin(xf * div)
    c_t = jnp.cos(yf * div)
    p_t = jnp.concatenate([s_t, c_t], axis=0)

    objh_t = jnp.dot(w1o_ref[...], oft_ref[...],
                     preferred_element_type=jnp.float32)
    blk_obj = oft_ref.shape[1]
    blk_pts = ys_ref.shape[2]
    row = jax.lax.broadcasted_iota(jnp.int32, (blk_obj, blk_pts), 0)
    col = jax.lax.broadcasted_iota(jnp.int32, (blk_obj, blk_pts), 1)
    e = jnp.where(row == col // g, 1.0, 0.0)

    h_t = (jnp.dot(w1p_ref[...], p_t, preferred_element_type=jnp.float32)
           + jnp.dot(objh_t, e, preferred_element_type=jnp.float32)
           + b1_ref[...])
    h_t = jnp.maximum(h_t, 0.0)

    out = jax.lax.dot_general(h_t, w2_ref[...],
                              (((0,), (1,)), ((), ())),
                              preferred_element_type=jnp.float32)
    feats_ref[...] = out + b2_ref[...]
    idx_ref[0] = y * width + x


def _scatter_body(idx_ref, feats_ref, bev_in_ref, bev_ref):
    del idx_ref, bev_in_ref
    bev_ref[...] = feats_ref[...]


def _scatter_rows(idx, feats, n_cells):
    n_pts, out_dim = feats.shape
    bev0 = jnp.zeros((n_cells, 1, out_dim), jnp.float32)
    grid_spec = pltpu.PrefetchScalarGridSpec(
        num_scalar_prefetch=1,
        grid=(n_pts,),
        in_specs=[
            pl.BlockSpec((1, 1, out_dim), lambda i, idx: (i, 0, 0)),
            pl.BlockSpec(memory_space=pl.ANY),
        ],
        out_specs=pl.BlockSpec((1, 1, out_dim), lambda i, idx: (idx[i], 0, 0)),
    )
    out = pl.pallas_call(
        _scatter_body,
        grid_spec=grid_spec,
        out_shape=jax.ShapeDtypeStruct((n_cells, 1, out_dim), jnp.float32),
        input_output_aliases={2: 0},
    )(idx, feats.reshape(n_pts, 1, out_dim), bev0)
    return out.reshape(n_cells, out_dim)


def kernel(object_grids, object_features, pos_encoding, W1, b1, W2, b2):
    h, w, _ = pos_encoding.shape
    out_dim = W2.shape[0]
    feats, idx = _compute_feats(object_grids, object_features, W1, b1, W2, b2,
                                width=w)
    bev = _scatter_rows(idx, feats, h * w)
    return bev.reshape(h, w, out_dim)


# speedup vs baseline: 59.4820x; 59.4820x over previous
import functools

import jax
import jax.numpy as jnp
import numpy as np
from jax import lax
from jax.experimental import pallas as pl
from jax.experimental.pallas import tpu as pltpu
from jax.experimental.pallas import tpu_sc as plsc


def _compute_feats(object_grids, object_features, W1, b1, W2, b2, *, width):
    n_obj, g, _ = object_grids.shape
    n_pts = n_obj * g
    pos_dim = W1.shape[1] - object_features.shape[1]
    hid = W1.shape[0]
    out_dim = W2.shape[0]

    n_blocks = 32
    blk_obj = n_obj // n_blocks
    blk_pts = blk_obj * g

    ys = object_grids[:, :, 0].reshape(n_blocks, 1, blk_pts)
    xs = object_grids[:, :, 1].reshape(n_blocks, 1, blk_pts)
    oft = jnp.swapaxes(
        object_features.reshape(n_blocks, blk_obj, object_features.shape[1]),
        1, 2).reshape(n_blocks * object_features.shape[1], blk_obj)

    w1p = jnp.concatenate([W1[:, 0:pos_dim:2], W1[:, 1:pos_dim:2]], axis=1)
    w1o = W1[:, pos_dim:]
    div = np.exp(np.arange(0, pos_dim, 2, dtype=np.float64)
                 * -(np.log(10000.0) / pos_dim)).astype(np.float32)
    div = jnp.asarray(div).reshape(pos_dim // 2, 1)

    grid = (n_blocks,)
    kernel_fn = functools.partial(_feats_kernel_body, g=g, width=width)
    feats, idx = pl.pallas_call(
        kernel_fn,
        grid=grid,
        in_specs=[
            pl.BlockSpec((pos_dim // 2, 1), lambda i: (0, 0)),
            pl.BlockSpec((1, 1, blk_pts), lambda i: (i, 0, 0)),
            pl.BlockSpec((1, 1, blk_pts), lambda i: (i, 0, 0)),
            pl.BlockSpec((object_features.shape[1], blk_obj), lambda i: (i, 0)),
            pl.BlockSpec((hid, pos_dim), lambda i: (0, 0)),
            pl.BlockSpec((hid, object_features.shape[1]), lambda i: (0, 0)),
            pl.BlockSpec((hid, 1), lambda i: (0, 0)),
            pl.BlockSpec((out_dim, hid), lambda i: (0, 0)),
            pl.BlockSpec((1, out_dim), lambda i: (0, 0)),
        ],
        out_specs=[
            pl.BlockSpec((blk_pts, out_dim), lambda i: (i, 0)),
            pl.BlockSpec((1, 1, blk_pts), lambda i: (i, 0, 0)),
        ],
        out_shape=[
            jax.ShapeDtypeStruct((n_pts, out_dim), jnp.float32),
            jax.ShapeDtypeStruct((n_blocks, 1, blk_pts), jnp.int32),
        ],
    )(div, ys, xs, oft, w1p, w1o, b1.reshape(hid, 1), W2,
      b2.reshape(1, out_dim))
    return feats, idx.reshape(n_pts)


def _feats_kernel_body(div_ref, ys_ref, xs_ref, oft_ref, w1p_ref, w1o_ref,
                       b1_ref, w2_ref, b2_ref, feats_ref, idx_ref, *, g, width):
    y = ys_ref[0]
    x = xs_ref[0]
    yf = y.astype(jnp.float32)
    xf = x.astype(jnp.float32)
    div = div_ref[...]
    s_t = jnp.sin(xf * div)
    c_t = jnp.cos(yf * div)
    p_t = jnp.concatenate([s_t, c_t], axis=0)

    objh_t = jnp.dot(w1o_ref[...], oft_ref[...],
                     preferred_element_type=jnp.float32)
    blk_obj = oft_ref.shape[1]
    blk_pts = ys_ref.shape[2]
    row = jax.lax.broadcasted_iota(jnp.int32, (blk_obj, blk_pts), 0)
    col = jax.lax.broadcasted_iota(jnp.int32, (blk_obj, blk_pts), 1)
    e = jnp.where(row == col // g, 1.0, 0.0)

    h_t = (jnp.dot(w1p_ref[...], p_t, preferred_element_type=jnp.float32)
           + jnp.dot(objh_t, e, preferred_element_type=jnp.float32)
           + b1_ref[...])
    h_t = jnp.maximum(h_t, 0.0)

    out = jax.lax.dot_general(h_t, w2_ref[...],
                              (((0,), (1,)), ((), ())),
                              preferred_element_type=jnp.float32)
    feats_ref[...] = out + b2_ref[...]
    idx_ref[0] = y * width + x


def _dyn_gather16(x, i):
    dn = lax.GatherDimensionNumbers(offset_dims=(), collapsed_slice_dims=(0,),
                                    start_index_map=(0,))
    return lax.gather(x, i[:, None], dn, (1,),
                      mode=lax.GatherScatterMode.PROMISE_IN_BOUNDS)


def _sc_scatter(feats, idx, n_cells):
    n_pts, out_dim = feats.shape
    n_workers = 32
    cells_w = n_cells // n_workers
    shift_w = cells_w.bit_length() - 1
    n_chunks = 8
    chunk = n_pts // n_chunks
    vregs_per_chunk = chunk // 16
    wtab = cells_w + 16
    zrows = 512
    dchunk = 128

    mesh = plsc.VectorSubcoreMesh(core_axis_name="c", subcore_axis_name="s")

    @functools.partial(
        pl.kernel,
        out_type=jax.ShapeDtypeStruct((n_cells, out_dim), jnp.float32),
        mesh=mesh,
        compiler_params=pltpu.CompilerParams(needs_layout_passes=False,
                                             use_tc_tiling_on_sc=False),
        scratch_types=[
            pltpu.VMEM((chunk,), jnp.int32),
            pltpu.VMEM((wtab,), jnp.int32),
            pltpu.VMEM((cells_w + dchunk,), jnp.int32),
            pltpu.VMEM((cells_w + dchunk,), jnp.int32),
            pltpu.VMEM((1, dchunk), jnp.int32),
            pltpu.VMEM((1, dchunk), jnp.int32),
            pltpu.VMEM((dchunk, out_dim), jnp.float32),
            pltpu.VMEM((zrows, out_dim), jnp.float32),
            pltpu.SemaphoreType.DMA,
            pltpu.SemaphoreType.DMA,
        ],
    )
    def sc_kernel(feats_hbm, idx_hbm, bev_hbm, ibuf, winner, win_pid, win_cell,
                  pid_row, cell_row, fbuf, zbuf, gsem, ssem):
        wid = lax.axis_index("c") * 16 + lax.axis_index("s")
        lane = lax.broadcasted_iota(jnp.int32, (16,), 0)
        zeros16 = jnp.zeros((16,), jnp.float32)

        def init_body(i, _):
            winner[pl.ds(i * 16, 16)] = jnp.full((16,), -1, jnp.int32)
            return 0
        lax.fori_loop(0, wtab // 16, init_body, 0, unroll=4)

        def zinit(i, _):
            for j in range(out_dim // 16):
                zbuf[i, pl.ds(j * 16, 16)] = zeros16
            return 0
        lax.fori_loop(0, zrows, zinit, 0, unroll=4)

        def chunk_body(c, _):
            pltpu.sync_copy(idx_hbm.at[pl.ds(c * chunk, chunk)], ibuf)

            def vreg_body(j, _):
                v = ibuf[pl.ds(j * 16, 16)]
                sel = (v >> shift_w) == wid
                lcell = jnp.where(sel, v & (cells_w - 1), cells_w + lane)
                pid = c * chunk + j * 16 + lane
                key = (lcell << 16) | pid
                ks, ps = plsc.sort_key_val(key, pid)
                cs = ks >> 16
                nxt = _dyn_gather16(cs, jnp.minimum(lane + 1, 15))
                mlast = (cs != nxt) | (lane == 15)
                plsc.store_scatter(winner, [cs], ps, mask=mlast)
                return 0
            lax.fori_loop(0, vregs_per_chunk, vreg_body, 0, unroll=4)
            return 0
        lax.fori_loop(0, n_chunks, chunk_body, 0)

        base_row = wid * cells_w
        def zero_body(k, _):
            pltpu.sync_copy(zbuf, bev_hbm.at[pl.ds(base_row + k * zrows,
                                                   zrows)])
            return 0
        lax.fori_loop(0, cells_w // zrows, zero_body, 0)

        def compact_body(t, cnt):
            wv = winner[pl.ds(t * 16, 16)]
            m = wv >= 0
            plsc.store_compressed(win_pid.at[pl.ds(cnt, 16)], wv, mask=m)
            cells_glob = base_row + t * 16 + lane
            plsc.store_compressed(win_cell.at[pl.ds(cnt, 16)], cells_glob,
                                  mask=m)
            return cnt + jnp.sum(m.astype(jnp.int32))
        nw = lax.fori_loop(0, cells_w // 16, compact_body, jnp.int32(0))

        @pl.when(nw > 0)
        def _pad():
            lastp = _dyn_gather16(win_pid[pl.ds(nw - 1, 16)],
                                  jnp.zeros((16,), jnp.int32))
            lastc = _dyn_gather16(win_cell[pl.ds(nw - 1, 16)],
                                  jnp.zeros((16,), jnp.int32))
            for j in range(dchunk // 16):
                win_pid[pl.ds(nw + j * 16, 16)] = lastp
                win_cell[pl.ds(nw + j * 16, 16)] = lastc

        n_dchunks = (nw + dchunk - 1) // dchunk
        def dma_body(t, _):
            for j in range(dchunk // 16):
                pid_row[0, pl.ds(j * 16, 16)] = win_pid[
                    pl.ds(t * dchunk + j * 16, 16)]
                cell_row[0, pl.ds(j * 16, 16)] = win_cell[
                    pl.ds(t * dchunk + j * 16, 16)]
            pltpu.async_copy(feats_hbm.at[pid_row.at[0]], fbuf, gsem).wait()
            pltpu.async_copy(fbuf, bev_hbm.at[cell_row.at[0]], ssem).wait()
            return 0
        lax.fori_loop(0, n_dchunks, dma_body, 0)

    return sc_kernel(feats, idx)


def kernel(object_grids, object_features, pos_encoding, W1, b1, W2, b2):
    h, w, _ = pos_encoding.shape
    out_dim = W2.shape[0]
    feats, idx = _compute_feats(object_grids, object_features, W1, b1, W2, b2,
                                width=w)
    bev = _sc_scatter(feats, idx, h * w)
    return bev.reshape(h, w, out_dim)


# speedup vs baseline: 59.7492x; 1.0045x over previous
import functools

import jax
import jax.numpy as jnp
import numpy as np
from jax import lax
from jax.experimental import pallas as pl
from jax.experimental.pallas import tpu as pltpu
from jax.experimental.pallas import tpu_sc as plsc


def _compute_feats(object_grids, object_features, W1, b1, W2, b2, *, width):
    n_obj, g, _ = object_grids.shape
    n_pts = n_obj * g
    pos_dim = W1.shape[1] - object_features.shape[1]
    hid = W1.shape[0]
    out_dim = W2.shape[0]

    n_blocks = 32
    blk_obj = n_obj // n_blocks
    blk_pts = blk_obj * g

    ys = object_grids[:, :, 0].reshape(n_blocks, 1, blk_pts)
    xs = object_grids[:, :, 1].reshape(n_blocks, 1, blk_pts)
    oft = jnp.swapaxes(
        object_features.reshape(n_blocks, blk_obj, object_features.shape[1]),
        1, 2).reshape(n_blocks * object_features.shape[1], blk_obj)

    w1p = jnp.concatenate([W1[:, 0:pos_dim:2], W1[:, 1:pos_dim:2]], axis=1)
    w1o = W1[:, pos_dim:]
    div = np.exp(np.arange(0, pos_dim, 2, dtype=np.float64)
                 * -(np.log(10000.0) / pos_dim)).astype(np.float32)
    div = jnp.asarray(div).reshape(pos_dim // 2, 1)

    grid = (n_blocks,)
    kernel_fn = functools.partial(_feats_kernel_body, g=g, width=width)
    feats, idx = pl.pallas_call(
        kernel_fn,
        grid=grid,
        in_specs=[
            pl.BlockSpec((pos_dim // 2, 1), lambda i: (0, 0)),
            pl.BlockSpec((1, 1, blk_pts), lambda i: (i, 0, 0)),
            pl.BlockSpec((1, 1, blk_pts), lambda i: (i, 0, 0)),
            pl.BlockSpec((object_features.shape[1], blk_obj), lambda i: (i, 0)),
            pl.BlockSpec((hid, pos_dim), lambda i: (0, 0)),
            pl.BlockSpec((hid, object_features.shape[1]), lambda i: (0, 0)),
            pl.BlockSpec((hid, 1), lambda i: (0, 0)),
            pl.BlockSpec((out_dim, hid), lambda i: (0, 0)),
            pl.BlockSpec((1, out_dim), lambda i: (0, 0)),
        ],
        out_specs=[
            pl.BlockSpec((blk_pts, out_dim), lambda i: (i, 0)),
            pl.BlockSpec((1, 1, blk_pts), lambda i: (i, 0, 0)),
        ],
        out_shape=[
            jax.ShapeDtypeStruct((n_pts, out_dim), jnp.float32),
            jax.ShapeDtypeStruct((n_blocks, 1, blk_pts), jnp.int32),
        ],
    )(div, ys, xs, oft, w1p, w1o, b1.reshape(hid, 1), W2,
      b2.reshape(1, out_dim))
    return feats, idx.reshape(n_pts)


def _feats_kernel_body(div_ref, ys_ref, xs_ref, oft_ref, w1p_ref, w1o_ref,
                       b1_ref, w2_ref, b2_ref, feats_ref, idx_ref, *, g, width):
    y = ys_ref[0]
    x = xs_ref[0]
    yf = y.astype(jnp.float32)
    xf = x.astype(jnp.float32)
    div = div_ref[...]
    s_t = jnp.sin(xf * div)
    c_t = jnp.cos(yf * div)
    p_t = jnp.concatenate([s_t, c_t], axis=0)

    objh_t = jnp.dot(w1o_ref[...], oft_ref[...],
                     preferred_element_type=jnp.float32)
    blk_obj = oft_ref.shape[1]
    blk_pts = ys_ref.shape[2]
    row = jax.lax.broadcasted_iota(jnp.int32, (blk_obj, blk_pts), 0)
    col = jax.lax.broadcasted_iota(jnp.int32, (blk_obj, blk_pts), 1)
    e = jnp.where(row == col // g, 1.0, 0.0)

    h_t = (jnp.dot(w1p_ref[...], p_t, preferred_element_type=jnp.float32)
           + jnp.dot(objh_t, e, preferred_element_type=jnp.float32)
           + b1_ref[...])
    h_t = jnp.maximum(h_t, 0.0)

    out = jax.lax.dot_general(h_t, w2_ref[...],
                              (((0,), (1,)), ((), ())),
                              preferred_element_type=jnp.float32)
    feats_ref[...] = out + b2_ref[...]
    idx_ref[0] = y * width + x


def _dyn_gather16(x, i):
    dn = lax.GatherDimensionNumbers(offset_dims=(), collapsed_slice_dims=(0,),
                                    start_index_map=(0,))
    return lax.gather(x, i[:, None], dn, (1,),
                      mode=lax.GatherScatterMode.PROMISE_IN_BOUNDS)


def _sc_scatter(feats, idx, n_cells):
    n_pts, out_dim = feats.shape
    n_workers = 32
    cells_w = n_cells // n_workers
    shift_w = cells_w.bit_length() - 1
    n_chunks = 8
    chunk = n_pts // n_chunks
    vregs_per_chunk = chunk // 16
    wtab = cells_w + 16
    zrows = 512
    dchunk = 128

    mesh = plsc.VectorSubcoreMesh(core_axis_name="c", subcore_axis_name="s")

    @functools.partial(
        pl.kernel,
        out_type=jax.ShapeDtypeStruct((n_cells, out_dim), jnp.float32),
        mesh=mesh,
        compiler_params=pltpu.CompilerParams(needs_layout_passes=False,
                                             use_tc_tiling_on_sc=False),
        scratch_types=[
            pltpu.VMEM((2, chunk), jnp.int32),
            pltpu.VMEM((wtab,), jnp.int32),
            pltpu.VMEM((cells_w + dchunk,), jnp.int32),
            pltpu.VMEM((cells_w + dchunk,), jnp.int32),
            pltpu.VMEM((1, dchunk), jnp.int32),
            pltpu.VMEM((1, dchunk), jnp.int32),
            pltpu.VMEM((dchunk, out_dim), jnp.float32),
            pltpu.VMEM((zrows, out_dim), jnp.float32),
            pltpu.SemaphoreType.DMA,
            pltpu.SemaphoreType.DMA,
            pltpu.SemaphoreType.DMA,
            pltpu.SemaphoreType.DMA,
        ],
    )
    def sc_kernel(feats_hbm, idx_hbm, bev_hbm, ibuf, winner, win_pid, win_cell,
                  pid_row, cell_row, fbuf, zbuf, gsem, ssem, zsem, isem):
        wid = lax.axis_index("c") * 16 + lax.axis_index("s")
        lane = lax.broadcasted_iota(jnp.int32, (16,), 0)
        zeros16 = jnp.zeros((16,), jnp.float32)

        def zinit(i, _):
            for j in range(out_dim // 16):
                zbuf[i, pl.ds(j * 16, 16)] = zeros16
            return 0
        lax.fori_loop(0, zrows, zinit, 0, unroll=4)

        base_row = wid * cells_w
        zcopies = []
        for k in range(cells_w // zrows):
            zcopies.append(pltpu.async_copy(
                zbuf, bev_hbm.at[pl.ds(base_row + k * zrows, zrows)], zsem))

        icopy = pltpu.async_copy(idx_hbm.at[pl.ds(0, chunk)], ibuf.at[0], isem)

        def init_body(i, _):
            winner[pl.ds(i * 16, 16)] = jnp.full((16,), -1, jnp.int32)
            return 0
        lax.fori_loop(0, wtab // 16, init_body, 0, unroll=4)

        for c in range(n_chunks):
            icopy.wait()
            if c + 1 < n_chunks:
                icopy = pltpu.async_copy(
                    idx_hbm.at[pl.ds((c + 1) * chunk, chunk)],
                    ibuf.at[(c + 1) % 2], isem)
            cbuf = ibuf.at[c % 2]

            def vreg_body(j, _, c=c, cbuf=cbuf):
                v = cbuf[pl.ds(j * 16, 16)]
                sel = (v >> shift_w) == wid
                lcell = jnp.where(sel, v & (cells_w - 1), cells_w + lane)
                pid = c * chunk + j * 16 + lane
                key = (lcell << 16) | pid
                ks, ps = plsc.sort_key_val(key, pid)
                cs = ks >> 16
                nxt = _dyn_gather16(cs, jnp.minimum(lane + 1, 15))
                mlast = (cs != nxt) | (lane == 15)
                plsc.store_scatter(winner, [cs], ps, mask=mlast)
                return 0
            lax.fori_loop(0, vregs_per_chunk, vreg_body, 0, unroll=4)

        for zc in zcopies:
            zc.wait()

        def compact_body(t, cnt):
            wv = winner[pl.ds(t * 16, 16)]
            m = wv >= 0
            plsc.store_compressed(win_pid.at[pl.ds(cnt, 16)], wv, mask=m)
            cells_glob = base_row + t * 16 + lane
            plsc.store_compressed(win_cell.at[pl.ds(cnt, 16)], cells_glob,
                                  mask=m)
            return cnt + jnp.sum(m.astype(jnp.int32))
        nw = lax.fori_loop(0, cells_w // 16, compact_body, jnp.int32(0))

        @pl.when(nw > 0)
        def _pad():
            lastp = _dyn_gather16(win_pid[pl.ds(nw - 1, 16)],
                                  jnp.zeros((16,), jnp.int32))
            lastc = _dyn_gather16(win_cell[pl.ds(nw - 1, 16)],
                                  jnp.zeros((16,), jnp.int32))
            for j in range(dchunk // 16):
                win_pid[pl.ds(nw + j * 16, 16)] = lastp
                win_cell[pl.ds(nw + j * 16, 16)] = lastc

        n_dchunks = (nw + dchunk - 1) // dchunk
        def dma_body(t, _):
            for j in range(dchunk // 16):
                pid_row[0, pl.ds(j * 16, 16)] = win_pid[
                    pl.ds(t * dchunk + j * 16, 16)]
                cell_row[0, pl.ds(j * 16, 16)] = win_cell[
                    pl.ds(t * dchunk + j * 16, 16)]
            pltpu.async_copy(feats_hbm.at[pid_row.at[0]], fbuf, gsem).wait()
            pltpu.async_copy(fbuf, bev_hbm.at[cell_row.at[0]], ssem).wait()
            return 0
        lax.fori_loop(0, n_dchunks, dma_body, 0)

    return sc_kernel(feats, idx)


def kernel(object_grids, object_features, pos_encoding, W1, b1, W2, b2):
    h, w, _ = pos_encoding.shape
    out_dim = W2.shape[0]
    feats, idx = _compute_feats(object_grids, object_features, W1, b1, W2, b2,
                                width=w)
    bev = _sc_scatter(feats, idx, h * w)
    return bev.reshape(h, w, out_dim)


# speedup vs baseline: 74.2750x; 1.2431x over previous
import functools

import jax
import jax.numpy as jnp
import numpy as np
from jax import lax
from jax.experimental import pallas as pl
from jax.experimental.pallas import tpu as pltpu
from jax.experimental.pallas import tpu_sc as plsc


def _compute_idx(ys, xs, *, width):
    n_blocks, _, blk_pts = ys.shape

    def body(ys_ref, xs_ref, idx_ref):
        idx_ref[0] = ys_ref[0] * width + xs_ref[0]

    idx = pl.pallas_call(
        body,
        grid=(n_blocks,),
        in_specs=[
            pl.BlockSpec((1, 1, blk_pts), lambda i: (i, 0, 0)),
            pl.BlockSpec((1, 1, blk_pts), lambda i: (i, 0, 0)),
        ],
        out_specs=pl.BlockSpec((1, 1, blk_pts), lambda i: (i, 0, 0)),
        out_shape=jax.ShapeDtypeStruct((n_blocks, 1, blk_pts), jnp.int32),
    )(ys, xs)
    return idx.reshape(n_blocks * blk_pts)


def _compute_feats(ys, xs, object_features, W1, b1, W2, b2):
    n_blocks, _, blk_pts = ys.shape
    n_pts = n_blocks * blk_pts
    g = 32
    blk_obj = blk_pts // g
    pos_dim = W1.shape[1] - object_features.shape[1]
    hid = W1.shape[0]
    out_dim = W2.shape[0]

    oft = jnp.swapaxes(
        object_features.reshape(n_blocks, blk_obj, object_features.shape[1]),
        1, 2).reshape(n_blocks * object_features.shape[1], blk_obj)

    w1p = jnp.concatenate([W1[:, 0:pos_dim:2], W1[:, 1:pos_dim:2]], axis=1)
    w1o = W1[:, pos_dim:]
    div = np.exp(np.arange(0, pos_dim, 2, dtype=np.float64)
                 * -(np.log(10000.0) / pos_dim)).astype(np.float32)
    div = jnp.asarray(div).reshape(pos_dim // 2, 1)

    grid = (n_blocks,)
    kernel_fn = functools.partial(_feats_kernel_body, g=g)
    feats = pl.pallas_call(
        kernel_fn,
        grid=grid,
        in_specs=[
            pl.BlockSpec((pos_dim // 2, 1), lambda i: (0, 0)),
            pl.BlockSpec((1, 1, blk_pts), lambda i: (i, 0, 0)),
            pl.BlockSpec((1, 1, blk_pts), lambda i: (i, 0, 0)),
            pl.BlockSpec((object_features.shape[1], blk_obj), lambda i: (i, 0)),
            pl.BlockSpec((hid, pos_dim), lambda i: (0, 0)),
            pl.BlockSpec((hid, object_features.shape[1]), lambda i: (0, 0)),
            pl.BlockSpec((hid, 1), lambda i: (0, 0)),
            pl.BlockSpec((out_dim, hid), lambda i: (0, 0)),
            pl.BlockSpec((1, out_dim), lambda i: (0, 0)),
        ],
        out_specs=pl.BlockSpec((blk_pts, out_dim), lambda i: (i, 0)),
        out_shape=jax.ShapeDtypeStruct((n_pts, out_dim), jnp.float32),
    )(div, ys, xs, oft, w1p, w1o, b1.reshape(hid, 1), W2,
      b2.reshape(1, out_dim))
    return feats


def _feats_kernel_body(div_ref, ys_ref, xs_ref, oft_ref, w1p_ref, w1o_ref,
                       b1_ref, w2_ref, b2_ref, feats_ref, *, g):
    y = ys_ref[0]
    x = xs_ref[0]
    yf = y.astype(jnp.float32)
    xf = x.astype(jnp.float32)
    div = div_ref[...]
    s_t = jnp.sin(xf * div)
    c_t = jnp.cos(yf * div)
    p_t = jnp.concatenate([s_t, c_t], axis=0)

    objh_t = jnp.dot(w1o_ref[...], oft_ref[...],
                     preferred_element_type=jnp.float32)
    blk_obj = oft_ref.shape[1]
    blk_pts = ys_ref.shape[2]
    row = jax.lax.broadcasted_iota(jnp.int32, (blk_obj, blk_pts), 0)
    col = jax.lax.broadcasted_iota(jnp.int32, (blk_obj, blk_pts), 1)
    e = jnp.where(row == col // g, 1.0, 0.0)

    h_t = (jnp.dot(w1p_ref[...], p_t, preferred_element_type=jnp.float32)
           + jnp.dot(objh_t, e, preferred_element_type=jnp.float32)
           + b1_ref[...])
    h_t = jnp.maximum(h_t, 0.0)

    out = jax.lax.dot_general(h_t, w2_ref[...],
                              (((0,), (1,)), ((), ())),
                              preferred_element_type=jnp.float32)
    feats_ref[...] = out + b2_ref[...]


def _dyn_gather16(x, i):
    dn = lax.GatherDimensionNumbers(offset_dims=(), collapsed_slice_dims=(0,),
                                    start_index_map=(0,))
    return lax.gather(x, i[:, None], dn, (1,),
                      mode=lax.GatherScatterMode.PROMISE_IN_BOUNDS)


def _sc_scan(idx, n_cells):
    n_pts = idx.shape[0]
    n_workers = 32
    cells_w = n_cells // n_workers
    shift_w = cells_w.bit_length() - 1
    n_chunks = 8
    chunk = n_pts // n_chunks
    vregs_per_chunk = chunk // 16
    wtab = cells_w + 16

    mesh = plsc.VectorSubcoreMesh(core_axis_name="c", subcore_axis_name="s")

    @functools.partial(
        pl.kernel,
        out_type=jax.ShapeDtypeStruct((n_cells,), jnp.int32),
        mesh=mesh,
        compiler_params=pltpu.CompilerParams(needs_layout_passes=False,
                                             use_tc_tiling_on_sc=False),
        scratch_types=[
            pltpu.VMEM((2, chunk), jnp.int32),
            pltpu.VMEM((wtab,), jnp.int32),
            pltpu.SemaphoreType.DMA,
        ],
    )
    def scan_kernel(idx_hbm, win_hbm, ibuf, winner, isem):
        wid = lax.axis_index("c") * 16 + lax.axis_index("s")
        lane = lax.broadcasted_iota(jnp.int32, (16,), 0)

        icopy = pltpu.async_copy(idx_hbm.at[pl.ds(0, chunk)], ibuf.at[0], isem)

        def init_body(i, _):
            winner[pl.ds(i * 16, 16)] = jnp.full((16,), -1, jnp.int32)
            return 0
        lax.fori_loop(0, wtab // 16, init_body, 0, unroll=4)

        for c in range(n_chunks):
            icopy.wait()
            if c + 1 < n_chunks:
                icopy = pltpu.async_copy(
                    idx_hbm.at[pl.ds((c + 1) * chunk, chunk)],
                    ibuf.at[(c + 1) % 2], isem)
            cbuf = ibuf.at[c % 2]

            def vreg_body(j, _, c=c, cbuf=cbuf):
                v = cbuf[pl.ds(j * 16, 16)]
                sel = (v >> shift_w) == wid
                lcell = jnp.where(sel, v & (cells_w - 1), cells_w + lane)
                pid = c * chunk + j * 16 + lane
                key = (lcell << 16) | pid
                ks, ps = plsc.sort_key_val(key, pid)
                cs = ks >> 16
                nxt = _dyn_gather16(cs, jnp.minimum(lane + 1, 15))
                mlast = (cs != nxt) | (lane == 15)
                plsc.store_scatter(winner, [cs], ps, mask=mlast)
                return 0
            lax.fori_loop(0, vregs_per_chunk, vreg_body, 0, unroll=4)

        pltpu.sync_copy(winner.at[pl.ds(0, cells_w)],
                        win_hbm.at[pl.ds(wid * cells_w, cells_w)])

    return scan_kernel(idx)


def _sc_scatter(feats, win, n_cells):
    n_pts, out_dim = feats.shape
    n_workers = 32
    cells_w = n_cells // n_workers
    zrows = 512
    dchunk = 128

    mesh = plsc.VectorSubcoreMesh(core_axis_name="c", subcore_axis_name="s")

    @functools.partial(
        pl.kernel,
        out_type=jax.ShapeDtypeStruct((n_cells, out_dim), jnp.float32),
        mesh=mesh,
        compiler_params=pltpu.CompilerParams(needs_layout_passes=False,
                                             use_tc_tiling_on_sc=False),
        scratch_types=[
            pltpu.VMEM((cells_w,), jnp.int32),
            pltpu.VMEM((cells_w + dchunk,), jnp.int32),
            pltpu.VMEM((cells_w + dchunk,), jnp.int32),
            pltpu.VMEM((1, dchunk), jnp.int32),
            pltpu.VMEM((1, dchunk), jnp.int32),
            pltpu.VMEM((dchunk, out_dim), jnp.float32),
            pltpu.VMEM((zrows, out_dim), jnp.float32),
            pltpu.SemaphoreType.DMA,
            pltpu.SemaphoreType.DMA,
            pltpu.SemaphoreType.DMA,
            pltpu.SemaphoreType.DMA,
        ],
    )
    def scatter_kernel(feats_hbm, win_hbm, bev_hbm, wbuf, win_pid, win_cell,
                       pid_row, cell_row, fbuf, zbuf, gsem, ssem, zsem, wsem):
        wid = lax.axis_index("c") * 16 + lax.axis_index("s")
        lane = lax.broadcasted_iota(jnp.int32, (16,), 0)
        zeros16 = jnp.zeros((16,), jnp.float32)
        base_row = wid * cells_w

        wcopy = pltpu.async_copy(win_hbm.at[pl.ds(base_row, cells_w)], wbuf,
                                 wsem)

        def zinit(i, _):
            for j in range(out_dim // 16):
                zbuf[i, pl.ds(j * 16, 16)] = zeros16
            return 0
        lax.fori_loop(0, zrows, zinit, 0, unroll=4)

        zcopies = []
        for k in range(cells_w // zrows):
            zcopies.append(pltpu.async_copy(
                zbuf, bev_hbm.at[pl.ds(base_row + k * zrows, zrows)], zsem))

        wcopy.wait()

        def compact_body(t, cnt):
            wv = wbuf[pl.ds(t * 16, 16)]
            m = wv >= 0
            plsc.store_compressed(win_pid.at[pl.ds(cnt, 16)], wv, mask=m)
            cells_glob = base_row + t * 16 + lane
            plsc.store_compressed(win_cell.at[pl.ds(cnt, 16)], cells_glob,
                                  mask=m)
            return cnt + jnp.sum(m.astype(jnp.int32))
        nw = lax.fori_loop(0, cells_w // 16, compact_body, jnp.int32(0))

        @pl.when(nw > 0)
        def _pad():
            lastp = _dyn_gather16(win_pid[pl.ds(nw - 1, 16)],
                                  jnp.zeros((16,), jnp.int32))
            lastc = _dyn_gather16(win_cell[pl.ds(nw - 1, 16)],
                                  jnp.zeros((16,), jnp.int32))
            for j in range(dchunk // 16):
                win_pid[pl.ds(nw + j * 16, 16)] = lastp
                win_cell[pl.ds(nw + j * 16, 16)] = lastc

        for zc in zcopies:
            zc.wait()

        n_dchunks = (nw + dchunk - 1) // dchunk
        def dma_body(t, _):
            for j in range(dchunk // 16):
                pid_row[0, pl.ds(j * 16, 16)] = win_pid[
                    pl.ds(t * dchunk + j * 16, 16)]
                cell_row[0, pl.ds(j * 16, 16)] = win_cell[
                    pl.ds(t * dchunk + j * 16, 16)]
            pltpu.async_copy(feats_hbm.at[pid_row.at[0]], fbuf, gsem).wait()
            pltpu.async_copy(fbuf, bev_hbm.at[cell_row.at[0]], ssem).wait()
            return 0
        lax.fori_loop(0, n_dchunks, dma_body, 0)

    return scatter_kernel(feats, win)


def kernel(object_grids, object_features, pos_encoding, W1, b1, W2, b2):
    h, w, _ = pos_encoding.shape
    out_dim = W2.shape[0]
    n_obj, g, _ = object_grids.shape
    n_blocks = 32
    blk_obj = n_obj // n_blocks
    blk_pts = blk_obj * g

    ys = object_grids[:, :, 0].reshape(n_blocks, 1, blk_pts)
    xs = object_grids[:, :, 1].reshape(n_blocks, 1, blk_pts)

    idx = _compute_idx(ys, xs, width=w)
    win = _sc_scan(idx, h * w)
    feats = _compute_feats(ys, xs, object_features, W1, b1, W2, b2)
    bev = _sc_scatter(feats, win, h * w)
    return bev.reshape(h, w, out_dim)


# speedup vs baseline: 79.0004x; 1.0636x over previous
import functools

import jax
import jax.numpy as jnp
import numpy as np
from jax import lax
from jax.experimental import pallas as pl
from jax.experimental.pallas import tpu as pltpu
from jax.experimental.pallas import tpu_sc as plsc


def _compute_idx(ys, xs, *, width):
    n_blocks, _, blk_pts = ys.shape

    def body(ys_ref, xs_ref, idx_ref):
        idx_ref[0] = ys_ref[0] * width + xs_ref[0]

    idx = pl.pallas_call(
        body,
        grid=(n_blocks,),
        in_specs=[
            pl.BlockSpec((1, 1, blk_pts), lambda i: (i, 0, 0)),
            pl.BlockSpec((1, 1, blk_pts), lambda i: (i, 0, 0)),
        ],
        out_specs=pl.BlockSpec((1, 1, blk_pts), lambda i: (i, 0, 0)),
        out_shape=jax.ShapeDtypeStruct((n_blocks, 1, blk_pts), jnp.int32),
    )(ys, xs)
    return idx.reshape(n_blocks * blk_pts)


def _compute_feats(ys, xs, object_features, W1, b1, W2, b2):
    n_blocks, _, blk_pts = ys.shape
    n_pts = n_blocks * blk_pts
    g = 32
    blk_obj = blk_pts // g
    pos_dim = W1.shape[1] - object_features.shape[1]
    hid = W1.shape[0]
    out_dim = W2.shape[0]

    oft = jnp.swapaxes(
        object_features.reshape(n_blocks, blk_obj, object_features.shape[1]),
        1, 2).reshape(n_blocks * object_features.shape[1], blk_obj)

    w1p = jnp.concatenate([W1[:, 0:pos_dim:2], W1[:, 1:pos_dim:2]], axis=1)
    w1o = W1[:, pos_dim:]
    div = np.exp(np.arange(0, pos_dim, 2, dtype=np.float64)
                 * -(np.log(10000.0) / pos_dim)) / (2.0 * np.pi)
    div = jnp.asarray(div.astype(np.float32)).reshape(pos_dim // 2, 1)

    grid = (n_blocks,)
    kernel_fn = functools.partial(_feats_kernel_body, g=g)
    feats = pl.pallas_call(
        kernel_fn,
        grid=grid,
        in_specs=[
            pl.BlockSpec((pos_dim // 2, 1), lambda i: (0, 0)),
            pl.BlockSpec((1, 1, blk_pts), lambda i: (i, 0, 0)),
            pl.BlockSpec((1, 1, blk_pts), lambda i: (i, 0, 0)),
            pl.BlockSpec((object_features.shape[1], blk_obj), lambda i: (i, 0)),
            pl.BlockSpec((hid, pos_dim), lambda i: (0, 0)),
            pl.BlockSpec((hid, object_features.shape[1]), lambda i: (0, 0)),
            pl.BlockSpec((hid, 1), lambda i: (0, 0)),
            pl.BlockSpec((out_dim, hid), lambda i: (0, 0)),
            pl.BlockSpec((1, out_dim), lambda i: (0, 0)),
        ],
        out_specs=pl.BlockSpec((blk_pts, out_dim), lambda i: (i, 0)),
        out_shape=jax.ShapeDtypeStruct((n_pts, out_dim), jnp.float32),
    )(div, ys, xs, oft, w1p, w1o, b1.reshape(hid, 1), W2,
      b2.reshape(1, out_dim))
    return feats


def _sin2pi(u):
    r = u - jnp.round(u)
    t = r * r
    p = jnp.float32(-12.37239574)
    p = p * t + jnp.float32(41.26987033)
    p = p * t + jnp.float32(-76.59491552)
    p = p * t + jnp.float32(81.59765671)
    p = p * t + jnp.float32(-41.34148031)
    p = p * t + jnp.float32(6.28318347)
    return r * p


def _feats_kernel_body(div_ref, ys_ref, xs_ref, oft_ref, w1p_ref, w1o_ref,
                       b1_ref, w2_ref, b2_ref, feats_ref, *, g):
    y = ys_ref[0]
    x = xs_ref[0]
    yf = y.astype(jnp.float32)
    xf = x.astype(jnp.float32)
    div = div_ref[...]
    s_t = _sin2pi(xf * div)
    c_t = _sin2pi(yf * div + jnp.float32(0.25))
    p_t = jnp.concatenate([s_t, c_t], axis=0)

    objh_t = jnp.dot(w1o_ref[...], oft_ref[...],
                     preferred_element_type=jnp.float32)
    blk_obj = oft_ref.shape[1]
    blk_pts = ys_ref.shape[2]
    row = jax.lax.broadcasted_iota(jnp.int32, (blk_obj, blk_pts), 0)
    col = jax.lax.broadcasted_iota(jnp.int32, (blk_obj, blk_pts), 1)
    e = jnp.where(row == col // g, 1.0, 0.0)

    h_t = (jnp.dot(w1p_ref[...], p_t, preferred_element_type=jnp.float32)
           + jnp.dot(objh_t, e, preferred_element_type=jnp.float32)
           + b1_ref[...])
    h_t = jnp.maximum(h_t, 0.0)

    out = jax.lax.dot_general(h_t, w2_ref[...],
                              (((0,), (1,)), ((), ())),
                              preferred_element_type=jnp.float32)
    feats_ref[...] = out + b2_ref[...]


def _dyn_gather16(x, i):
    dn = lax.GatherDimensionNumbers(offset_dims=(), collapsed_slice_dims=(0,),
                                    start_index_map=(0,))
    return lax.gather(x, i[:, None], dn, (1,),
                      mode=lax.GatherScatterMode.PROMISE_IN_BOUNDS)


def _sc_scan(idx, n_cells):
    n_pts = idx.shape[0]
    n_workers = 32
    cells_w = n_cells // n_workers
    shift_w = cells_w.bit_length() - 1
    n_chunks = 8
    chunk = n_pts // n_chunks
    vregs_per_chunk = chunk // 16
    wtab = cells_w + 16

    mesh = plsc.VectorSubcoreMesh(core_axis_name="c", subcore_axis_name="s")

    @functools.partial(
        pl.kernel,
        out_type=jax.ShapeDtypeStruct((n_cells,), jnp.int32),
        mesh=mesh,
        compiler_params=pltpu.CompilerParams(needs_layout_passes=False,
                                             use_tc_tiling_on_sc=False),
        scratch_types=[
            pltpu.VMEM((2, chunk), jnp.int32),
            pltpu.VMEM((wtab,), jnp.int32),
            pltpu.SemaphoreType.DMA,
        ],
    )
    def scan_kernel(idx_hbm, win_hbm, ibuf, winner, isem):
        wid = lax.axis_index("c") * 16 + lax.axis_index("s")
        lane = lax.broadcasted_iota(jnp.int32, (16,), 0)

        icopy = pltpu.async_copy(idx_hbm.at[pl.ds(0, chunk)], ibuf.at[0], isem)

        def init_body(i, _):
            winner[pl.ds(i * 16, 16)] = jnp.full((16,), -1, jnp.int32)
            return 0
        lax.fori_loop(0, wtab // 16, init_body, 0, unroll=4)

        for c in range(n_chunks):
            icopy.wait()
            if c + 1 < n_chunks:
                icopy = pltpu.async_copy(
                    idx_hbm.at[pl.ds((c + 1) * chunk, chunk)],
                    ibuf.at[(c + 1) % 2], isem)
            cbuf = ibuf.at[c % 2]

            def vreg_body(j, _, c=c, cbuf=cbuf):
                v = cbuf[pl.ds(j * 16, 16)]
                sel = (v >> shift_w) == wid
                lcell = jnp.where(sel, v & (cells_w - 1), cells_w + lane)
                pid = c * chunk + j * 16 + lane
                key = (lcell << 16) | pid
                ks, ps = plsc.sort_key_val(key, pid)
                cs = ks >> 16
                nxt = _dyn_gather16(cs, jnp.minimum(lane + 1, 15))
                mlast = (cs != nxt) | (lane == 15)
                plsc.store_scatter(winner, [cs], ps, mask=mlast)
                return 0
            lax.fori_loop(0, vregs_per_chunk, vreg_body, 0, unroll=4)

        pltpu.sync_copy(winner.at[pl.ds(0, cells_w)],
                        win_hbm.at[pl.ds(wid * cells_w, cells_w)])

    return scan_kernel(idx)


def _sc_scatter(feats, win, n_cells):
    n_pts, out_dim = feats.shape
    n_workers = 32
    cells_w = n_cells // n_workers
    zrows = 512
    dchunk = 128

    mesh = plsc.VectorSubcoreMesh(core_axis_name="c", subcore_axis_name="s")

    @functools.partial(
        pl.kernel,
        out_type=jax.ShapeDtypeStruct((n_cells, out_dim), jnp.float32),
        mesh=mesh,
        compiler_params=pltpu.CompilerParams(needs_layout_passes=False,
                                             use_tc_tiling_on_sc=False),
        scratch_types=[
            pltpu.VMEM((cells_w,), jnp.int32),
            pltpu.VMEM((cells_w + dchunk,), jnp.int32),
            pltpu.VMEM((cells_w + dchunk,), jnp.int32),
            pltpu.VMEM((1, dchunk), jnp.int32),
            pltpu.VMEM((1, dchunk), jnp.int32),
            pltpu.VMEM((dchunk, out_dim), jnp.float32),
            pltpu.VMEM((zrows, out_dim), jnp.float32),
            pltpu.SemaphoreType.DMA,
            pltpu.SemaphoreType.DMA,
            pltpu.SemaphoreType.DMA,
            pltpu.SemaphoreType.DMA,
        ],
    )
    def scatter_kernel(feats_hbm, win_hbm, bev_hbm, wbuf, win_pid, win_cell,
                       pid_row, cell_row, fbuf, zbuf, gsem, ssem, zsem, wsem):
        wid = lax.axis_index("c") * 16 + lax.axis_index("s")
        lane = lax.broadcasted_iota(jnp.int32, (16,), 0)
        zeros16 = jnp.zeros((16,), jnp.float32)
        base_row = wid * cells_w

        wcopy = pltpu.async_copy(win_hbm.at[pl.ds(base_row, cells_w)], wbuf,
                                 wsem)

        def zinit(i, _):
            for j in range(out_dim // 16):
                zbuf[i, pl.ds(j * 16, 16)] = zeros16
            return 0
        lax.fori_loop(0, zrows, zinit, 0, unroll=4)

        zcopies = []
        for k in range(cells_w // zrows):
            zcopies.append(pltpu.async_copy(
                zbuf, bev_hbm.at[pl.ds(base_row + k * zrows, zrows)], zsem))

        wcopy.wait()

        def compact_body(t, cnt):
            wv = wbuf[pl.ds(t * 16, 16)]
            m = wv >= 0
            plsc.store_compressed(win_pid.at[pl.ds(cnt, 16)], wv, mask=m)
            cells_glob = base_row + t * 16 + lane
            plsc.store_compressed(win_cell.at[pl.ds(cnt, 16)], cells_glob,
                                  mask=m)
            return cnt + jnp.sum(m.astype(jnp.int32))
        nw = lax.fori_loop(0, cells_w // 16, compact_body, jnp.int32(0))

        @pl.when(nw > 0)
        def _pad():
            lastp = _dyn_gather16(win_pid[pl.ds(nw - 1, 16)],
                                  jnp.zeros((16,), jnp.int32))
            lastc = _dyn_gather16(win_cell[pl.ds(nw - 1, 16)],
                                  jnp.zeros((16,), jnp.int32))
            for j in range(dchunk // 16):
                win_pid[pl.ds(nw + j * 16, 16)] = lastp
                win_cell[pl.ds(nw + j * 16, 16)] = lastc

        for zc in zcopies:
            zc.wait()

        n_dchunks = (nw + dchunk - 1) // dchunk
        def dma_body(t, _):
            for j in range(dchunk // 16):
                pid_row[0, pl.ds(j * 16, 16)] = win_pid[
                    pl.ds(t * dchunk + j * 16, 16)]
                cell_row[0, pl.ds(j * 16, 16)] = win_cell[
                    pl.ds(t * dchunk + j * 16, 16)]
            pltpu.async_copy(feats_hbm.at[pid_row.at[0]], fbuf, gsem).wait()
            pltpu.async_copy(fbuf, bev_hbm.at[cell_row.at[0]], ssem).wait()
            return 0
        lax.fori_loop(0, n_dchunks, dma_body, 0)

    return scatter_kernel(feats, win)


def kernel(object_grids, object_features, pos_encoding, W1, b1, W2, b2):
    h, w, _ = pos_encoding.shape
    out_dim = W2.shape[0]
    n_obj, g, _ = object_grids.shape
    n_blocks = 32
    blk_obj = n_obj // n_blocks
    blk_pts = blk_obj * g

    ys = object_grids[:, :, 0].reshape(n_blocks, 1, blk_pts)
    xs = object_grids[:, :, 1].reshape(n_blocks, 1, blk_pts)

    idx = _compute_idx(ys, xs, width=w)
    win = _sc_scan(idx, h * w)
    feats = _compute_feats(ys, xs, object_features, W1, b1, W2, b2)
    bev = _sc_scatter(feats, win, h * w)
    return bev.reshape(h, w, out_dim)
